# baseline (speedup 1.0000x reference)
import functools

import jax
import jax.numpy as jnp
from jax import lax
from jax.experimental import pallas as pl
from jax.experimental.pallas import tpu as pltpu
from jax.experimental.pallas import tpu_sc as plsc

_NUM_CORES = 2
_NUM_SUBCORES = 16
_NW = _NUM_CORES * _NUM_SUBCORES
_LANES = 16


@functools.partial(jax.jit, static_argnames=("n_rows", "seq_len", "d"))
def _emb_lookup(idx_flat, token_table, pos_table, *, n_rows, seq_len, d):
    rows_per_w = n_rows // _NW
    chunk = 2 * seq_len
    n_chunks = rows_per_w // chunk
    d_vregs = d // _LANES

    mesh = plsc.VectorSubcoreMesh(core_axis_name="c", subcore_axis_name="s")

    @functools.partial(
        pl.kernel,
        mesh=mesh,
        compiler_params=pltpu.CompilerParams(use_tc_tiling_on_sc=False),
        out_type=jax.ShapeDtypeStruct((n_rows, d), jnp.float32),
        scratch_types=[
            pltpu.VMEM((chunk,), jnp.int32),
            pltpu.VMEM((chunk, d), jnp.float32),
            pltpu.VMEM((chunk, d), jnp.float32),
            pltpu.SemaphoreType.DMA,
        ],
    )
    def body(idx_hbm, tok_hbm, pos_hbm, out_hbm, idx_v, rows_v, pos_v, sem):
        cid = lax.axis_index("c")
        sid = lax.axis_index("s")
        wid = sid * _NUM_CORES + cid
        base = wid * rows_per_w

        pltpu.sync_copy(pos_hbm, pos_v.at[pl.ds(0, seq_len)])
        pltpu.sync_copy(pos_hbm, pos_v.at[pl.ds(seq_len, seq_len)])

        def chunk_body(g, carry):
            start = base + g * chunk
            pltpu.sync_copy(idx_hbm.at[pl.ds(start, chunk)], idx_v)
            pltpu.async_copy(tok_hbm.at[idx_v], rows_v, sem).wait()

            def row_body(i, c2):
                for cc in range(d_vregs):
                    sl = pl.ds(cc * _LANES, _LANES)
                    rows_v[i, sl] = rows_v[i, sl] + pos_v[i, sl]
                return c2

            lax.fori_loop(0, chunk, row_body, 0, unroll=2)
            pltpu.sync_copy(rows_v, out_hbm.at[pl.ds(start, chunk)])
            return carry

        lax.fori_loop(0, n_chunks, chunk_body, 0)

    return body(idx_flat, token_table, pos_table)


def kernel(inputs, token_table, pos_table):
    b, s = inputs.shape
    d = token_table.shape[1]
    idx_flat = inputs.reshape(-1).astype(jnp.int32)
    out = _emb_lookup(idx_flat, token_table, pos_table,
                      n_rows=b * s, seq_len=s, d=d)
    return out.reshape(b, s, d)


# speedup vs baseline: 1.0674x; 1.0674x over previous
import functools

import jax
import jax.numpy as jnp
from jax import lax
from jax.experimental import pallas as pl
from jax.experimental.pallas import tpu as pltpu
from jax.experimental.pallas import tpu_sc as plsc

_NUM_CORES = 2
_NUM_SUBCORES = 16
_NW = _NUM_CORES * _NUM_SUBCORES
_LANES = 16
_CHUNK_SEQS = 2


@functools.partial(jax.jit, static_argnames=("n_rows", "seq_len", "d"))
def _emb_lookup(idx_flat, token_table, pos_table, *, n_rows, seq_len, d):
    rows_per_w = n_rows // _NW
    chunk = _CHUNK_SEQS * seq_len
    n_chunks = rows_per_w // chunk
    n_pairs = n_chunks // 2
    d_vregs = d // _LANES

    mesh = plsc.VectorSubcoreMesh(core_axis_name="c", subcore_axis_name="s")

    @functools.partial(
        pl.kernel,
        mesh=mesh,
        compiler_params=pltpu.CompilerParams(use_tc_tiling_on_sc=False),
        out_type=jax.ShapeDtypeStruct((n_rows, d), jnp.float32),
        scratch_types=[
            pltpu.VMEM((chunk,), jnp.int32),
            pltpu.VMEM((chunk,), jnp.int32),
            pltpu.VMEM((chunk, d), jnp.float32),
            pltpu.VMEM((chunk, d), jnp.float32),
            pltpu.VMEM((seq_len, d), jnp.float32),
            pltpu.SemaphoreType.DMA,
            pltpu.SemaphoreType.DMA,
            pltpu.SemaphoreType.DMA,
            pltpu.SemaphoreType.DMA,
        ],
    )
    def body(idx_hbm, tok_hbm, pos_hbm, out_hbm,
             idx_a, idx_b, rows_a, rows_b, pos_v,
             gsem_a, gsem_b, ssem_a, ssem_b):
        cid = lax.axis_index("c")
        sid = lax.axis_index("s")
        wid = sid * _NUM_CORES + cid
        base = wid * rows_per_w

        pltpu.sync_copy(pos_hbm, pos_v)

        def start_gather(g, idx_r, rows_r, gsem):
            start = base + g * chunk
            pltpu.sync_copy(idx_hbm.at[pl.ds(start, chunk)], idx_r)
            pltpu.async_copy(tok_hbm.at[idx_r], rows_r, gsem)

        def wait_gather(idx_r, rows_r, gsem):
            pltpu.make_async_copy(tok_hbm.at[idx_r], rows_r, gsem).wait()

        def add_pos(rows_r):
            def seg_body(i, carry):
                for s2 in range(_CHUNK_SEQS):
                    r = s2 * seq_len + i
                    for cc in range(d_vregs):
                        sl = pl.ds(cc * _LANES, _LANES)
                        rows_r[r, sl] = rows_r[r, sl] + pos_v[i, sl]
                return carry
            lax.fori_loop(0, seq_len, seg_body, 0, unroll=8)

        def start_store(g, rows_r, ssem):
            start = base + g * chunk
            pltpu.async_copy(rows_r, out_hbm.at[pl.ds(start, chunk)], ssem)

        def wait_store(g, rows_r, ssem):
            start = base + g * chunk
            pltpu.make_async_copy(
                rows_r, out_hbm.at[pl.ds(start, chunk)], ssem).wait()

        start_gather(0, idx_a, rows_a, gsem_a)

        def pair_body(h, carry):
            ga = 2 * h
            gb = 2 * h + 1

            @pl.when(h > 0)
            def _():
                wait_store(gb - 2, rows_b, ssem_b)
            start_gather(gb, idx_b, rows_b, gsem_b)

            wait_gather(idx_a, rows_a, gsem_a)
            add_pos(rows_a)
            start_store(ga, rows_a, ssem_a)

            @pl.when(h + 1 < n_pairs)
            def _():
                wait_store(ga, rows_a, ssem_a)
                start_gather(ga + 2, idx_a, rows_a, gsem_a)

            wait_gather(idx_b, rows_b, gsem_b)
            add_pos(rows_b)
            start_store(gb, rows_b, ssem_b)
            return carry

        lax.fori_loop(0, n_pairs, pair_body, 0)

        wait_store(n_chunks - 2, rows_a, ssem_a)
        wait_store(n_chunks - 1, rows_b, ssem_b)

    return body(idx_flat, token_table, pos_table)


def kernel(inputs, token_table, pos_table):
    b, s = inputs.shape
    d = token_table.shape[1]
    idx_flat = inputs.reshape(-1).astype(jnp.int32)
    out = _emb_lookup(idx_flat, token_table, pos_table,
                      n_rows=b * s, seq_len=s, d=d)
    return out.reshape(b, s, d)
